# baseline (speedup 1.0000x reference)
import functools

import jax
import jax.numpy as jnp
from jax import lax
from jax.experimental import pallas as pl
from jax.experimental.pallas import tpu as pltpu
from jax.experimental.pallas import tpu_sc as plsc

N = 10000
E = 320000
D = 128
NC = 2
NS = 16
NW = NC * NS
CHUNK = 128
EPW = 10240
E_PAD = NW * EPW
NCHUNK = EPW // CHUNK
NBUF = 2
G = 10
NGRP = NCHUNK // G
N_PAD = 10240
DUMP_ROW = N
RPS = N_PAD // NS

_sc_mesh = plsc.VectorSubcoreMesh(
    core_axis_name="c", subcore_axis_name="s", num_cores=NC, num_subcores=NS
)


@functools.partial(
    pl.kernel,
    out_type=jax.ShapeDtypeStruct((NC, N_PAD, D), jnp.float32),
    mesh=_sc_mesh,
    scratch_types=[
        pltpu.VMEM((2, G, CHUNK), jnp.int32),
        pltpu.VMEM((2, G, CHUNK), jnp.int32),
        pltpu.VMEM((NBUF, CHUNK, D), jnp.float32),
        pltpu.VMEM_SHARED((N_PAD, D), jnp.float32),
        [pltpu.SemaphoreType.DMA] * NBUF,
        [pltpu.SemaphoreType.DMA] * 2,
    ],
)
def _propagate(hw_hbm, src_hbm, dst_hbm, zero_hbm, p_hbm,
               src_v, dst_v, rows_v, acc_sh, gsems, isems):
    c = lax.axis_index("c")
    s = lax.axis_index("s")
    wid = c * NS + s
    pltpu.sync_copy(src_hbm.at[wid].at[0], src_v.at[0])
    pltpu.sync_copy(dst_hbm.at[wid].at[0], dst_v.at[0])
    pltpu.sync_copy(zero_hbm, acc_sh.at[pl.ds(s * RPS, RPS)])
    plsc.subcore_barrier()

    for b in range(NBUF):
        pltpu.async_copy(hw_hbm.at[src_v.at[0].at[b]], rows_v.at[b],
                         gsems[b])

    @pl.loop(0, NGRP)
    def _(g):
        cur = lax.rem(g, 2)
        nxt_slot = 1 - cur
        for k in range(G):
            b = k % NBUF
            if k == NBUF:
                pltpu.async_copy(src_hbm.at[wid].at[g + 1],
                                 src_v.at[nxt_slot], isems[0])
                pltpu.async_copy(dst_hbm.at[wid].at[g + 1],
                                 dst_v.at[nxt_slot], isems[1])
            if k == G - NBUF:
                pltpu.make_async_copy(src_hbm.at[wid].at[g + 1],
                                      src_v.at[nxt_slot], isems[0]).wait()
                pltpu.make_async_copy(dst_hbm.at[wid].at[g + 1],
                                      dst_v.at[nxt_slot], isems[1]).wait()
            pltpu.make_async_copy(hw_hbm.at[src_v.at[cur].at[k]],
                                  rows_v.at[b], gsems[b]).wait()
            pltpu.sync_copy(rows_v.at[b], acc_sh.at[dst_v.at[cur].at[k]],
                            add=True)
            kn = k + NBUF
            if kn < G:
                pltpu.async_copy(hw_hbm.at[src_v.at[cur].at[kn]],
                                 rows_v.at[b], gsems[b])
            else:
                pltpu.async_copy(hw_hbm.at[src_v.at[nxt_slot].at[kn - G]],
                                 rows_v.at[b], gsems[b])

    for b in range(NBUF):
        pltpu.make_async_copy(hw_hbm.at[src_v.at[0].at[b]],
                              rows_v.at[b], gsems[b]).wait()

    plsc.subcore_barrier()
    pltpu.sync_copy(acc_sh.at[pl.ds(s * RPS, RPS)],
                    p_hbm.at[c].at[pl.ds(s * RPS, RPS)])


ROWS_BLK = 1000


def _mm_first_body(x_ref, w_ref, o_ref):
    o_ref[...] = jnp.dot(x_ref[...], w_ref[...],
                         preferred_element_type=jnp.float32)


def _mm_fused_body(p0_ref, p1_ref, b_ref, w_ref, o_ref):
    h = jnp.maximum(p0_ref[...] + p1_ref[...] + b_ref[...], 0.0)
    o_ref[...] = jnp.dot(h, w_ref[...], preferred_element_type=jnp.float32)


def _final_body(p0_ref, p1_ref, b_ref, o_ref):
    o_ref[...] = p0_ref[...] + p1_ref[...] + b_ref[...]


def _mm_first(x, w):
    return pl.pallas_call(
        _mm_first_body,
        grid=(N // ROWS_BLK,),
        in_specs=[
            pl.BlockSpec((ROWS_BLK, D), lambda i: (i, 0)),
            pl.BlockSpec((D, D), lambda i: (0, 0)),
        ],
        out_specs=pl.BlockSpec((ROWS_BLK, D), lambda i: (i, 0)),
        out_shape=jax.ShapeDtypeStruct((N, D), jnp.float32),
    )(x, w)


def _mm_fused(p, b, w):
    return pl.pallas_call(
        _mm_fused_body,
        grid=(N // ROWS_BLK,),
        in_specs=[
            pl.BlockSpec((ROWS_BLK, D), lambda i: (i, 0)),
            pl.BlockSpec((ROWS_BLK, D), lambda i: (i, 0)),
            pl.BlockSpec((1, D), lambda i: (0, 0)),
            pl.BlockSpec((D, D), lambda i: (0, 0)),
        ],
        out_specs=pl.BlockSpec((ROWS_BLK, D), lambda i: (i, 0)),
        out_shape=jax.ShapeDtypeStruct((N, D), jnp.float32),
    )(p[0], p[1], b, w)


def _final(p, b):
    return pl.pallas_call(
        _final_body,
        grid=(N // ROWS_BLK,),
        in_specs=[
            pl.BlockSpec((ROWS_BLK, D), lambda i: (i, 0)),
            pl.BlockSpec((ROWS_BLK, D), lambda i: (i, 0)),
            pl.BlockSpec((1, D), lambda i: (0, 0)),
        ],
        out_specs=pl.BlockSpec((ROWS_BLK, D), lambda i: (i, 0)),
        out_shape=jax.ShapeDtypeStruct((N, D), jnp.float32),
    )(p[0], p[1], b)


def kernel(x, edge_index, W1, b1, W2, b2, W3, b3):
    pad = E_PAD - E
    src = jnp.concatenate(
        [edge_index[0].astype(jnp.int32), jnp.zeros((pad,), jnp.int32)]
    ).reshape(NW, NGRP, G * CHUNK)
    src = jnp.concatenate(
        [src, jnp.zeros((NW, 1, G * CHUNK), jnp.int32)], axis=1
    ).reshape(NW, NGRP + 1, G, CHUNK)
    dst = jnp.concatenate(
        [edge_index[1].astype(jnp.int32),
         jnp.full((pad,), DUMP_ROW, jnp.int32)]
    ).reshape(NW, NGRP, G * CHUNK)
    dst = jnp.concatenate(
        [dst, jnp.full((NW, 1, G * CHUNK), DUMP_ROW, jnp.int32)], axis=1
    ).reshape(NW, NGRP + 1, G, CHUNK)
    zeros = jnp.zeros((RPS, D), jnp.float32)
    b1r = b1.reshape(1, D)
    b2r = b2.reshape(1, D)
    b3r = b3.reshape(1, D)

    h = _mm_first(x, W1)
    p = _propagate(h, src, dst, zeros)
    h = _mm_fused(p, b1r, W2)
    p = _propagate(h, src, dst, zeros)
    h = _mm_fused(p, b2r, W3)
    p = _propagate(h, src, dst, zeros)
    return _final(p, b3r)


# speedup vs baseline: 4.1854x; 4.1854x over previous
import functools

import jax
import jax.numpy as jnp
from jax import lax
from jax.experimental import pallas as pl
from jax.experimental.pallas import tpu as pltpu
from jax.experimental.pallas import tpu_sc as plsc

N = 10000
E = 320000
D = 128
NC = 2
NS = 16
NW = NC * NS
EPW = E // NW
CHUNK = 80
NCHUNK = EPW // CHUNK
N_PAD = 10240
RPS = N_PAD // NS

_sc_mesh = plsc.VectorSubcoreMesh(
    core_axis_name="c", subcore_axis_name="s", num_cores=NC, num_subcores=NS
)


@functools.partial(
    pl.kernel,
    out_type=jax.ShapeDtypeStruct((NC, N_PAD, D), jnp.float32),
    mesh=_sc_mesh,
    scratch_types=[
        pltpu.VMEM((EPW,), jnp.int32),
        pltpu.VMEM((NCHUNK, CHUNK), jnp.int32),
        pltpu.VMEM((2, CHUNK, D), jnp.float32),
        pltpu.VMEM_SHARED((N_PAD, D), jnp.float32),
        pltpu.SemaphoreType.DMA,
        pltpu.SemaphoreType.DMA,
    ],
)
def _propagate(hw_hbm, src_hbm, dst_hbm, zero_hbm, p_hbm,
               src_v, dst_v, rows_v, acc_sh, sem0, sem1):
    c = lax.axis_index("c")
    s = lax.axis_index("s")
    wid = c * NS + s
    pltpu.sync_copy(src_hbm.at[wid], src_v)
    pltpu.sync_copy(dst_hbm.at[wid], dst_v)
    pltpu.sync_copy(zero_hbm, acc_sh.at[pl.ds(s * RPS, RPS)])
    plsc.subcore_barrier()

    @pl.loop(0, NCHUNK - 1, step=2)
    def _(j):
        h0 = pltpu.async_copy(
            hw_hbm.at[src_v.at[pl.ds(j * CHUNK, CHUNK)]], rows_v.at[0],
            sem0)
        h1 = pltpu.async_copy(
            hw_hbm.at[src_v.at[pl.ds((j + 1) * CHUNK, CHUNK)]],
            rows_v.at[1], sem1)
        h0.wait()
        pltpu.sync_copy(rows_v.at[0], acc_sh.at[dst_v.at[j]], add=True)
        h1.wait()
        pltpu.sync_copy(rows_v.at[1], acc_sh.at[dst_v.at[j + 1]], add=True)

    hl = pltpu.async_copy(
        hw_hbm.at[src_v.at[pl.ds((NCHUNK - 1) * CHUNK, CHUNK)]],
        rows_v.at[0], sem0)
    hl.wait()
    pltpu.sync_copy(rows_v.at[0], acc_sh.at[dst_v.at[NCHUNK - 1]], add=True)

    plsc.subcore_barrier()
    pltpu.sync_copy(acc_sh.at[pl.ds(s * RPS, RPS)],
                    p_hbm.at[c].at[pl.ds(s * RPS, RPS)])


ROWS_BLK = 1000


def _mm_first_body(x_ref, w_ref, o_ref):
    o_ref[...] = jnp.dot(x_ref[...], w_ref[...],
                         preferred_element_type=jnp.float32)


def _mm_fused_body(p0_ref, p1_ref, b_ref, w_ref, o_ref):
    h = jnp.maximum(p0_ref[...] + p1_ref[...] + b_ref[...], 0.0)
    o_ref[...] = jnp.dot(h, w_ref[...], preferred_element_type=jnp.float32)


def _final_body(p0_ref, p1_ref, b_ref, o_ref):
    o_ref[...] = p0_ref[...] + p1_ref[...] + b_ref[...]


def _mm_first(x, w):
    return pl.pallas_call(
        _mm_first_body,
        grid=(N // ROWS_BLK,),
        in_specs=[
            pl.BlockSpec((ROWS_BLK, D), lambda i: (i, 0)),
            pl.BlockSpec((D, D), lambda i: (0, 0)),
        ],
        out_specs=pl.BlockSpec((ROWS_BLK, D), lambda i: (i, 0)),
        out_shape=jax.ShapeDtypeStruct((N, D), jnp.float32),
    )(x, w)


def _mm_fused(p, b, w):
    return pl.pallas_call(
        _mm_fused_body,
        grid=(N // ROWS_BLK,),
        in_specs=[
            pl.BlockSpec((ROWS_BLK, D), lambda i: (i, 0)),
            pl.BlockSpec((ROWS_BLK, D), lambda i: (i, 0)),
            pl.BlockSpec((1, D), lambda i: (0, 0)),
            pl.BlockSpec((D, D), lambda i: (0, 0)),
        ],
        out_specs=pl.BlockSpec((ROWS_BLK, D), lambda i: (i, 0)),
        out_shape=jax.ShapeDtypeStruct((N, D), jnp.float32),
    )(p[0], p[1], b, w)


def _final(p, b):
    return pl.pallas_call(
        _final_body,
        grid=(N // ROWS_BLK,),
        in_specs=[
            pl.BlockSpec((ROWS_BLK, D), lambda i: (i, 0)),
            pl.BlockSpec((ROWS_BLK, D), lambda i: (i, 0)),
            pl.BlockSpec((1, D), lambda i: (0, 0)),
        ],
        out_specs=pl.BlockSpec((ROWS_BLK, D), lambda i: (i, 0)),
        out_shape=jax.ShapeDtypeStruct((N, D), jnp.float32),
    )(p[0], p[1], b)


def kernel(x, edge_index, W1, b1, W2, b2, W3, b3):
    src = edge_index[0].astype(jnp.int32).reshape(NW, EPW)
    dst = edge_index[1].astype(jnp.int32).reshape(NW, NCHUNK, CHUNK)
    zeros = jnp.zeros((RPS, D), jnp.float32)
    b1r = b1.reshape(1, D)
    b2r = b2.reshape(1, D)
    b3r = b3.reshape(1, D)

    h = _mm_first(x, W1)
    p = _propagate(h, src, dst, zeros)
    h = _mm_fused(p, b1r, W2)
    p = _propagate(h, src, dst, zeros)
    h = _mm_fused(p, b2r, W3)
    p = _propagate(h, src, dst, zeros)
    return _final(p, b3r)
